# baseline (speedup 1.0000x reference)
import jax
import jax.numpy as jnp
from jax import lax
from jax.experimental import pallas as pl
from jax.experimental.pallas import tpu as pltpu
from jax.experimental.pallas import tpu_sc as plsc

N_NODES = 10000
N_EDGES = 320000
D = 128
NC = 2
NS = 16
NW = NC * NS
B = 128
EPW = 10240
NB = EPW // B
PAD_DST = 10111
ROWS_PER_TILE = 632
ACC_ROWS = NS * ROWS_PER_TILE
DEG_ROWS = 80


def _sc_body(src_hbm, dst_hbm, ew_hbm, em_hbm, feat_hbm,
             out_hbm, outd_hbm,
             acc, accd,
             srcb0, srcb1, srcb2, dstb0, dstb1, dstb2,
             ewb0, ewb1, ewb2, emb0, emb1, emb2,
             gbuf0, gbuf1, hist, rowidx, idmat,
             gsem0, gsem1, isem0, isem1, isem2, ssem0, ssem1, tsem):
    c = lax.axis_index("c")
    s = lax.axis_index("s")
    wid = c * NS + s
    ebase = wid * EPW

    zeros16 = jnp.zeros((16,), jnp.float32)
    ones16 = jnp.ones((16,), jnp.float32)
    iota16 = lax.broadcasted_iota(jnp.int32, (16,), 0)
    gbufs = (gbuf0, gbuf1)
    gsems = (gsem0, gsem1)
    ssems = (ssem0, ssem1)
    isems = (isem0, isem1, isem2)
    srcbs = (srcb0, srcb1, srcb2)
    dstbs = (dstb0, dstb1, dstb2)
    ewbs = (ewb0, ewb1, ewb2)
    embs = (emb0, emb1, emb2)

    for g in range(DEG_ROWS // 16):
        rowidx[pl.ds(g * 16, 16)] = iota16 + g * 16

    def _zero_row(r, carry):
        for cc in range(D // 16):
            gbuf0[r, pl.ds(cc * 16, 16)] = zeros16
        return carry
    lax.fori_loop(0, B, _zero_row, 0)

    def _zero_hist(r, carry):
        for cc in range(D // 16):
            hist[r, pl.ds(cc * 16, 16)] = zeros16
        return carry
    lax.fori_loop(0, DEG_ROWS, _zero_hist, 0)

    tile_base = s * ROWS_PER_TILE
    for k in range(4):
        pltpu.sync_copy(gbuf0, acc.at[pl.ds(tile_base + k * B, B)])
    pltpu.sync_copy(gbuf0.at[pl.ds(0, 120)],
                    acc.at[pl.ds(tile_base + 512, 120)])

    @pl.when(s == 0)
    def _zero_deg():
        pltpu.sync_copy(gbuf0.at[pl.ds(0, DEG_ROWS)], accd)

    plsc.subcore_barrier()

    def _issue_idx(i, b3):
        base = ebase + i * B
        pltpu.async_copy(src_hbm.at[pl.ds(base, B)], srcbs[b3], isems[b3])
        pltpu.async_copy(dst_hbm.at[pl.ds(base, B)], dstbs[b3], isems[b3])
        pltpu.async_copy(ew_hbm.at[pl.ds(base, B)], ewbs[b3], isems[b3])
        pltpu.async_copy(em_hbm.at[pl.ds(base, B)], embs[b3], isems[b3])

    def _drain_idx(i, b3):
        base = ebase + i * B
        pltpu.make_async_copy(src_hbm.at[pl.ds(base, B)], srcbs[b3], isems[b3]).wait()
        pltpu.make_async_copy(dst_hbm.at[pl.ds(base, B)], dstbs[b3], isems[b3]).wait()
        pltpu.make_async_copy(ew_hbm.at[pl.ds(base, B)], ewbs[b3], isems[b3]).wait()
        pltpu.make_async_copy(em_hbm.at[pl.ds(base, B)], embs[b3], isems[b3]).wait()

    def _compute(b2, b3):
        gbuf = gbufs[b2]

        def _group(g, gcarry):
            sl16 = pl.ds(g * 16, 16)
            wv16 = ewbs[b3][sl16] * embs[b3][sl16]
            dv16 = dstbs[b3][sl16]
            plsc.addupdate_scatter(
                hist,
                [lax.shift_right_logical(dv16, 7),
                 lax.bitwise_and(dv16, 127)],
                ones16)
            for k in range(16):
                wb = jnp.full((16,), wv16[k], jnp.float32)
                r = g * 16 + k
                for cc in range(D // 16):
                    sl = pl.ds(cc * 16, 16)
                    gbuf[r, sl] = gbuf[r, sl] * wb
            return gcarry
        lax.fori_loop(0, B // 16, _group, 0)

    def _batch(i, k, first=False, last=False):
        b2, b3 = k % 2, k % 3
        pb2, pb3 = (k + 1) % 2, (k + 2) % 3
        pltpu.make_async_copy(feat_hbm.at[srcbs[b3]], gbufs[b2], gsems[b2]).wait()
        if not first:
            pltpu.make_async_copy(gbufs[pb2], acc.at[dstbs[pb3]],
                                  ssems[pb2]).wait()
        if not last:
            _drain_idx(i + 1, (k + 1) % 3)
            pltpu.async_copy(feat_hbm.at[srcbs[(k + 1) % 3]], gbufs[pb2],
                             gsems[pb2])
        _compute(b2, b3)
        pltpu.async_copy(gbufs[b2], acc.at[dstbs[b3]], ssems[b2], add=True)
        if not last:
            nxt = i + 2
            if isinstance(nxt, int):
                if nxt < NB:
                    _issue_idx(nxt, pb3)
            else:
                @pl.when(nxt < NB)
                def _():
                    _issue_idx(nxt, pb3)

    _issue_idx(0, 0)
    _drain_idx(0, 0)
    _issue_idx(1, 1)
    pltpu.async_copy(feat_hbm.at[srcb0], gbuf0, gsem0)

    _batch(0, 0, first=True)
    for k in range(1, 6):
        _batch(k, k)

    def _six(p, carry):
        for k in range(6):
            _batch(6 * p + k, k)
        return carry
    lax.fori_loop(1, NB // 6, _six, 0)

    _batch(78, 0)
    _batch(79, 1, last=True)
    pltpu.make_async_copy(gbufs[1], acc.at[dstbs[1]], ssems[1]).wait()
    pltpu.async_copy(hist, accd.at[rowidx], tsem, add=True).wait()

    plsc.subcore_barrier()

    pltpu.sync_copy(acc.at[pl.ds(tile_base, ROWS_PER_TILE)],
                    out_hbm.at[c, pl.ds(tile_base, ROWS_PER_TILE)])

    @pl.when(s == 0)
    def _copy_deg():
        pltpu.sync_copy(accd, outd_hbm.at[c])


def _sc_aggregate(src, dst, ew, em, feat):
    mesh = plsc.VectorSubcoreMesh(core_axis_name="c", subcore_axis_name="s")
    k = pl.kernel(
        _sc_body,
        mesh=mesh,
        compiler_params=pltpu.CompilerParams(needs_layout_passes=False),
        out_type=(
            jax.ShapeDtypeStruct((NC, ACC_ROWS, D), jnp.float32),
            jax.ShapeDtypeStruct((NC, DEG_ROWS, D), jnp.float32),
        ),
        scratch_types=[
            pltpu.VMEM_SHARED((ACC_ROWS, D), jnp.float32),
            pltpu.VMEM_SHARED((DEG_ROWS, D), jnp.float32),
            pltpu.VMEM((B,), jnp.int32),
            pltpu.VMEM((B,), jnp.int32),
            pltpu.VMEM((B,), jnp.int32),
            pltpu.VMEM((B,), jnp.int32),
            pltpu.VMEM((B,), jnp.int32),
            pltpu.VMEM((B,), jnp.int32),
            pltpu.VMEM((B,), jnp.float32),
            pltpu.VMEM((B,), jnp.float32),
            pltpu.VMEM((B,), jnp.float32),
            pltpu.VMEM((B,), jnp.float32),
            pltpu.VMEM((B,), jnp.float32),
            pltpu.VMEM((B,), jnp.float32),
            pltpu.VMEM((B, D), jnp.float32),
            pltpu.VMEM((B, D), jnp.float32),
            pltpu.VMEM((DEG_ROWS, D), jnp.float32),
            pltpu.VMEM((DEG_ROWS,), jnp.int32),
            pltpu.VMEM((16, 16), jnp.float32),
            pltpu.SemaphoreType.DMA,
            pltpu.SemaphoreType.DMA,
            pltpu.SemaphoreType.DMA,
            pltpu.SemaphoreType.DMA,
            pltpu.SemaphoreType.DMA,
            pltpu.SemaphoreType.DMA,
            pltpu.SemaphoreType.DMA,
            pltpu.SemaphoreType.DMA,
        ],
    )
    return k(src, dst, ew, em, feat)


BM = 1280


def _tc_finish_body(acc_ref, deg_ref, feat_ref, wst_ref, wnt_ref, b_ref, out_ref):
    msg = acc_ref[0] + acc_ref[1]
    deg = (deg_ref[0] + deg_ref[1]).reshape(BM, 1)
    h = msg / jnp.maximum(deg, 1.0)
    out_ref[...] = (
        jnp.dot(feat_ref[...], wst_ref[...], preferred_element_type=jnp.float32)
        + jnp.dot(h, wnt_ref[...], preferred_element_type=jnp.float32)
        + b_ref[...]
    )


def _tc_finish(acc, deg, feat, wst, wnt, b):
    grid = ((N_NODES + BM - 1) // BM,)
    return pl.pallas_call(
        _tc_finish_body,
        grid=grid,
        in_specs=[
            pl.BlockSpec((NC, BM, D), lambda i: (0, i, 0)),
            pl.BlockSpec((NC, BM), lambda i: (0, i)),
            pl.BlockSpec((BM, D), lambda i: (i, 0)),
            pl.BlockSpec((D, D), lambda i: (0, 0)),
            pl.BlockSpec((D, D), lambda i: (0, 0)),
            pl.BlockSpec((1, D), lambda i: (0, 0)),
        ],
        out_specs=pl.BlockSpec((BM, D), lambda i: (i, 0)),
        out_shape=jax.ShapeDtypeStruct((N_NODES, D), jnp.float32),
    )(acc, deg, feat, wst, wnt, b)


def _pad_edges(x, fill):
    x = x.reshape(NW, N_EDGES // NW)
    return jnp.pad(x, ((0, 0), (0, EPW - N_EDGES // NW)),
                   constant_values=fill).reshape(-1)


def _pad_edges_dst(x):
    x = x.reshape(NW, N_EDGES // NW)
    npad = EPW - N_EDGES // NW
    padvals = 10000 + (jnp.arange(npad, dtype=jnp.int32) % 112)
    pad = jnp.broadcast_to(padvals, (NW, npad))
    return jnp.concatenate([x, pad], axis=1).reshape(-1)


def kernel(feat, edge_index, edge_weight, edge_mask,
           W_self, b_self, W_neigh, b_neigh):
    src = _pad_edges(edge_index[0].astype(jnp.int32), 0)
    dst = _pad_edges_dst(edge_index[1].astype(jnp.int32))
    ew = _pad_edges(edge_weight.reshape(-1), 0.0)
    em = _pad_edges(edge_mask.reshape(-1), 0.0)
    acc, deg = _sc_aggregate(src, dst, ew, em, feat)
    deg = deg.reshape(NC, DEG_ROWS * D)
    b = (b_self + b_neigh).reshape(1, D)
    return _tc_finish(acc, deg, feat, W_self.T, W_neigh.T, b)


# speedup vs baseline: 2.1867x; 2.1867x over previous
import jax
import jax.numpy as jnp
from jax import lax
from jax.experimental import pallas as pl
from jax.experimental.pallas import tpu as pltpu
from jax.experimental.pallas import tpu_sc as plsc

N_NODES = 10000
N_EDGES = 320000
D = 128
NC = 2
NS = 16
NW = NC * NS
B = 80
EPW = N_EDGES // NW
NB = EPW // B
ROWS_PER_TILE = 632
ACC_ROWS = NS * ROWS_PER_TILE
DEG_ROWS = 80


def _sc_body(src_hbm, dst_hbm, ew_hbm, em_hbm, feat_hbm,
             out_hbm, outd_hbm,
             acc, accd,
             srcb0, srcb1, srcb2, dstb0, dstb1, dstb2,
             ewb0, ewb1, ewb2, emb0, emb1, emb2,
             gbuf0, gbuf1, hist, rowidx, idmat,
             gsem0, gsem1, isem0, isem1, isem2, ssem0, ssem1, tsem):
    c = lax.axis_index("c")
    s = lax.axis_index("s")
    wid = c * NS + s
    ebase = wid * EPW

    zeros16 = jnp.zeros((16,), jnp.float32)
    ones16 = jnp.ones((16,), jnp.float32)
    iota16 = lax.broadcasted_iota(jnp.int32, (16,), 0)
    gbufs = (gbuf0, gbuf1)
    gsems = (gsem0, gsem1)
    ssems = (ssem0, ssem1)
    isems = (isem0, isem1, isem2)
    srcbs = (srcb0, srcb1, srcb2)
    dstbs = (dstb0, dstb1, dstb2)
    ewbs = (ewb0, ewb1, ewb2)
    embs = (emb0, emb1, emb2)

    for g in range(DEG_ROWS // 16):
        rowidx[pl.ds(g * 16, 16)] = iota16 + g * 16

    def _zero_row(r, carry):
        for cc in range(D // 16):
            gbuf0[r, pl.ds(cc * 16, 16)] = zeros16
        return carry
    lax.fori_loop(0, B, _zero_row, 0)

    def _zero_hist(r, carry):
        for cc in range(D // 16):
            hist[r, pl.ds(cc * 16, 16)] = zeros16
        return carry
    lax.fori_loop(0, DEG_ROWS, _zero_hist, 0)

    tile_base = s * ROWS_PER_TILE
    for k in range(7):
        pltpu.sync_copy(gbuf0, acc.at[pl.ds(tile_base + k * B, B)])
    pltpu.sync_copy(gbuf0.at[pl.ds(0, 72)], acc.at[pl.ds(tile_base + 560, 72)])

    @pl.when(s == 0)
    def _zero_deg():
        pltpu.sync_copy(gbuf0, accd)

    plsc.subcore_barrier()

    def _issue_idx(i, b3):
        base = ebase + i * B
        pltpu.async_copy(src_hbm.at[pl.ds(base, B)], srcbs[b3], isems[b3])
        pltpu.async_copy(dst_hbm.at[pl.ds(base, B)], dstbs[b3], isems[b3])
        pltpu.async_copy(ew_hbm.at[pl.ds(base, B)], ewbs[b3], isems[b3])
        pltpu.async_copy(em_hbm.at[pl.ds(base, B)], embs[b3], isems[b3])

    def _drain_idx(i, b3):
        base = ebase + i * B
        pltpu.make_async_copy(src_hbm.at[pl.ds(base, B)], srcbs[b3], isems[b3]).wait()
        pltpu.make_async_copy(dst_hbm.at[pl.ds(base, B)], dstbs[b3], isems[b3]).wait()
        pltpu.make_async_copy(ew_hbm.at[pl.ds(base, B)], ewbs[b3], isems[b3]).wait()
        pltpu.make_async_copy(em_hbm.at[pl.ds(base, B)], embs[b3], isems[b3]).wait()

    def _compute(b2, b3):
        gbuf = gbufs[b2]

        def _group(g, gcarry):
            sl16 = pl.ds(g * 16, 16)
            wv16 = ewbs[b3][sl16] * embs[b3][sl16]
            dv16 = dstbs[b3][sl16]
            plsc.addupdate_scatter(
                hist,
                [lax.shift_right_logical(dv16, 7),
                 lax.bitwise_and(dv16, 127)],
                ones16)
            for k in range(16):
                wb = jnp.full((16,), wv16[k], jnp.float32)
                r = g * 16 + k
                for cc in range(D // 16):
                    sl = pl.ds(cc * 16, 16)
                    gbuf[r, sl] = gbuf[r, sl] * wb
            return gcarry
        lax.fori_loop(0, B // 16, _group, 0)

    def _batch(i, k, first=False, last=False):
        b2, b3 = k % 2, k % 3
        pb2, pb3 = (k + 1) % 2, (k + 2) % 3
        pltpu.make_async_copy(feat_hbm.at[srcbs[b3]], gbufs[b2], gsems[b2]).wait()
        if not first:
            pltpu.make_async_copy(gbufs[pb2], acc.at[dstbs[pb3]],
                                  ssems[pb2]).wait()
        if not last:
            _drain_idx(i + 1, (k + 1) % 3)
            pltpu.async_copy(feat_hbm.at[srcbs[(k + 1) % 3]], gbufs[pb2],
                             gsems[pb2])
        _compute(b2, b3)
        pltpu.async_copy(gbufs[b2], acc.at[dstbs[b3]], ssems[b2], add=True)
        if not last:
            nxt = i + 2
            if isinstance(nxt, int):
                if nxt < NB:
                    _issue_idx(nxt, pb3)
            else:
                @pl.when(nxt < NB)
                def _():
                    _issue_idx(nxt, pb3)

    _issue_idx(0, 0)
    _drain_idx(0, 0)
    _issue_idx(1, 1)
    pltpu.async_copy(feat_hbm.at[srcb0], gbuf0, gsem0)

    _batch(0, 0, first=True)
    for k in range(1, 6):
        _batch(k, k)

    def _six(p, carry):
        for k in range(6):
            _batch(6 * p + k, k)
        return carry
    lax.fori_loop(1, NB // 6, _six, 0)

    for k in range(4):
        _batch(120 + k, k)
    _batch(124, 4, last=True)
    pltpu.make_async_copy(gbufs[0], acc.at[dstbs[1]], ssems[0]).wait()
    pltpu.async_copy(hist, accd.at[rowidx], tsem, add=True).wait()

    plsc.subcore_barrier()

    pltpu.sync_copy(acc.at[pl.ds(tile_base, ROWS_PER_TILE)],
                    out_hbm.at[c, pl.ds(tile_base, ROWS_PER_TILE)])

    @pl.when(s == 0)
    def _copy_deg():
        pltpu.sync_copy(accd, outd_hbm.at[c])


def _sc_aggregate(src, dst, ew, em, feat):
    mesh = plsc.VectorSubcoreMesh(core_axis_name="c", subcore_axis_name="s")
    k = pl.kernel(
        _sc_body,
        mesh=mesh,
        compiler_params=pltpu.CompilerParams(needs_layout_passes=False),
        out_type=(
            jax.ShapeDtypeStruct((NC, ACC_ROWS, D), jnp.float32),
            jax.ShapeDtypeStruct((NC, DEG_ROWS, D), jnp.float32),
        ),
        scratch_types=[
            pltpu.VMEM_SHARED((ACC_ROWS, D), jnp.float32),
            pltpu.VMEM_SHARED((DEG_ROWS, D), jnp.float32),
            pltpu.VMEM((B,), jnp.int32),
            pltpu.VMEM((B,), jnp.int32),
            pltpu.VMEM((B,), jnp.int32),
            pltpu.VMEM((B,), jnp.int32),
            pltpu.VMEM((B,), jnp.int32),
            pltpu.VMEM((B,), jnp.int32),
            pltpu.VMEM((B,), jnp.float32),
            pltpu.VMEM((B,), jnp.float32),
            pltpu.VMEM((B,), jnp.float32),
            pltpu.VMEM((B,), jnp.float32),
            pltpu.VMEM((B,), jnp.float32),
            pltpu.VMEM((B,), jnp.float32),
            pltpu.VMEM((B, D), jnp.float32),
            pltpu.VMEM((B, D), jnp.float32),
            pltpu.VMEM((DEG_ROWS, D), jnp.float32),
            pltpu.VMEM((DEG_ROWS,), jnp.int32),
            pltpu.VMEM((16, 16), jnp.float32),
            pltpu.SemaphoreType.DMA,
            pltpu.SemaphoreType.DMA,
            pltpu.SemaphoreType.DMA,
            pltpu.SemaphoreType.DMA,
            pltpu.SemaphoreType.DMA,
            pltpu.SemaphoreType.DMA,
            pltpu.SemaphoreType.DMA,
            pltpu.SemaphoreType.DMA,
        ],
    )
    return k(src, dst, ew, em, feat)


BM = 1280


def _tc_finish_body(acc_ref, deg_ref, feat_ref, wst_ref, wnt_ref, b_ref, out_ref):
    msg = acc_ref[0] + acc_ref[1]
    deg = (deg_ref[0] + deg_ref[1]).reshape(BM, 1)
    h = msg / jnp.maximum(deg, 1.0)
    out_ref[...] = (
        jnp.dot(feat_ref[...], wst_ref[...], preferred_element_type=jnp.float32)
        + jnp.dot(h, wnt_ref[...], preferred_element_type=jnp.float32)
        + b_ref[...]
    )


def _tc_finish(acc, deg, feat, wst, wnt, b):
    grid = ((N_NODES + BM - 1) // BM,)
    return pl.pallas_call(
        _tc_finish_body,
        grid=grid,
        in_specs=[
            pl.BlockSpec((NC, BM, D), lambda i: (0, i, 0)),
            pl.BlockSpec((NC, BM), lambda i: (0, i)),
            pl.BlockSpec((BM, D), lambda i: (i, 0)),
            pl.BlockSpec((D, D), lambda i: (0, 0)),
            pl.BlockSpec((D, D), lambda i: (0, 0)),
            pl.BlockSpec((1, D), lambda i: (0, 0)),
        ],
        out_specs=pl.BlockSpec((BM, D), lambda i: (i, 0)),
        out_shape=jax.ShapeDtypeStruct((N_NODES, D), jnp.float32),
    )(acc, deg, feat, wst, wnt, b)


def kernel(feat, edge_index, edge_weight, edge_mask,
           W_self, b_self, W_neigh, b_neigh):
    src = edge_index[0].astype(jnp.int32)
    dst = edge_index[1].astype(jnp.int32)
    ew = edge_weight.reshape(-1)
    em = edge_mask.reshape(-1)
    acc, deg = _sc_aggregate(src, dst, ew, em, feat)
    deg = deg.reshape(NC, DEG_ROWS * D)
    b = (b_self + b_neigh).reshape(1, D)
    return _tc_finish(acc, deg, feat, W_self.T, W_neigh.T, b)
